# Initial kernel scaffold; baseline (speedup 1.0000x reference)
import jax
import jax.numpy as jnp
from jax.experimental import pallas as pl


def kernel(input_ids, table):
    raise NotImplementedError("write your pallas kernel here")


# speedup vs baseline: 2.7048x; 2.7048x over previous
import functools
import hashlib

import jax
import jax.numpy as jnp
import numpy as np
from jax import lax
from jax.experimental import pallas as pl
from jax.experimental.pallas import tpu as pltpu
from jax.experimental.pallas import tpu_sc as plsc

NUM_BUCKETS = 100000
EMBED_DIM = 32
VOCAB = 1000000

TOKENS = 4096 * 50
NC, NS = 2, 16
NW = NC * NS
CHUNK = 128
ROWS_PER_W = TOKENS // NW // CHUNK
TOK_PER_W = TOKENS // NW
GROUP = 10
NGROUPS = ROWS_PER_W // GROUP


def _build_lut() -> np.ndarray:
    lut = np.empty((VOCAB,), dtype=np.int32)
    for t in range(VOCAB):
        h = hashlib.md5(str(t).encode()).hexdigest()
        lut[t] = int(h, 16) % NUM_BUCKETS
    return lut


_LUT = _build_lut()


def _make_sc_kernel():
    mesh = plsc.VectorSubcoreMesh(core_axis_name="c", subcore_axis_name="s")

    @functools.partial(
        pl.kernel,
        mesh=mesh,
        compiler_params=pltpu.CompilerParams(use_tc_tiling_on_sc=False),
        out_type=jax.ShapeDtypeStruct((TOKENS, EMBED_DIM), jnp.float32),
        scratch_types=[
            pltpu.VMEM((ROWS_PER_W, CHUNK), jnp.int32),
            pltpu.VMEM((ROWS_PER_W, CHUNK), jnp.int32),
            pltpu.VMEM((2, GROUP * CHUNK, EMBED_DIM), jnp.float32),
            pltpu.SemaphoreType.DMA,
            pltpu.SemaphoreType.DMA,
            pltpu.SemaphoreType.DMA,
        ],
    )
    def k(ids_hbm, lut_hbm, table_hbm, out_hbm, ids_v, hashed_v, rows_v,
          sem1, sem2, sem3):
        wid = lax.axis_index("s") * NC + lax.axis_index("c")

        pltpu.sync_copy(ids_hbm.at[wid], ids_v)

        def fire_lut(j, c):
            pltpu.async_copy(lut_hbm.at[ids_v.at[j]], hashed_v.at[j], sem1)
            return c
        lax.fori_loop(0, ROWS_PER_W, fire_lut, 0)
        pltpu.make_async_copy(ids_hbm.at[0], hashed_v, sem1).wait()

        out0 = wid * TOK_PER_W

        def group(g, c):
            buf = lax.rem(g, 2)

            @pl.when(g >= 2)
            def _():
                pltpu.make_async_copy(
                    out_hbm.at[pl.ds(0, GROUP * CHUNK)], rows_v.at[buf],
                    sem3).wait()

            def fire_rows(j, c2):
                pltpu.async_copy(
                    table_hbm.at[hashed_v.at[g * GROUP + j]],
                    rows_v.at[buf].at[pl.ds(j * CHUNK, CHUNK)], sem2)
                return c2
            lax.fori_loop(0, GROUP, fire_rows, 0)
            pltpu.make_async_copy(
                out_hbm.at[pl.ds(0, GROUP * CHUNK)], rows_v.at[buf],
                sem2).wait()
            pltpu.async_copy(
                rows_v.at[buf],
                out_hbm.at[pl.ds(out0 + g * GROUP * CHUNK, GROUP * CHUNK)],
                sem3)
            return c
        lax.fori_loop(0, NGROUPS, group, 0)

        pltpu.make_async_copy(out_hbm.at[pl.ds(0, GROUP * CHUNK)],
                              rows_v.at[0], sem3).wait()
        pltpu.make_async_copy(out_hbm.at[pl.ds(0, GROUP * CHUNK)],
                              rows_v.at[1], sem3).wait()

    return k


_sc_lookup = _make_sc_kernel()


def kernel(input_ids, table):
    ids3d = input_ids.reshape(NW, ROWS_PER_W, CHUNK)
    out = _sc_lookup(ids3d, jnp.asarray(_LUT), table)
    return out.reshape(*input_ids.shape, EMBED_DIM)


# speedup vs baseline: 4.4308x; 1.6382x over previous
import functools
import hashlib

import jax
import jax.numpy as jnp
import numpy as np
from jax import lax
from jax.experimental import pallas as pl
from jax.experimental.pallas import tpu as pltpu
from jax.experimental.pallas import tpu_sc as plsc

NUM_BUCKETS = 100000
EMBED_DIM = 32
VOCAB = 1000000

BATCH, SEQ = 4096, 50
NC, NS = 2, 16
NW = NC * NS
ROWS_PER_W = BATCH // NW
TOK_PER_W = ROWS_PER_W * SEQ
LANES = 16

CHUNK = 80
NCHUNK = TOK_PER_W // CHUNK
S1G = 20
NS1G = NCHUNK // S1G
GROUPC = 10
GROUP_TOK = GROUPC * CHUNK
GROUP_ROWS = GROUP_TOK // SEQ
NGROUPS = NCHUNK // GROUPC


def _build_lut() -> np.ndarray:
    lut = np.empty((VOCAB,), dtype=np.int32)
    for t in range(VOCAB):
        h = hashlib.md5(str(t).encode()).hexdigest()
        lut[t] = int(h, 16) % NUM_BUCKETS
    return lut


_LUT = _build_lut()

_T = np.arange(TOK_PER_W, dtype=np.int32)
_ROWIDX = _T // SEQ
_COLIDX = _T % SEQ


def _make_sc_kernel():
    mesh = plsc.VectorSubcoreMesh(core_axis_name="c", subcore_axis_name="s")

    @functools.partial(
        pl.kernel,
        mesh=mesh,
        compiler_params=pltpu.CompilerParams(use_tc_tiling_on_sc=False,
                                             needs_layout_passes=False),
        out_type=jax.ShapeDtypeStruct((BATCH, SEQ, EMBED_DIM), jnp.float32),
        scratch_types=[
            pltpu.VMEM((ROWS_PER_W, SEQ), jnp.int32),
            pltpu.VMEM((TOK_PER_W,), jnp.int32),
            pltpu.VMEM((TOK_PER_W,), jnp.int32),
            pltpu.VMEM((TOK_PER_W,), jnp.int32),
            pltpu.VMEM((TOK_PER_W,), jnp.int32),
            pltpu.VMEM((2, GROUP_TOK, EMBED_DIM), jnp.float32),
            pltpu.SemaphoreType.DMA,
            pltpu.SemaphoreType.DMA,
            pltpu.SemaphoreType.DMA,
        ],
    )
    def k(ids_hbm, lut_hbm, table_hbm, rowidx_hbm, colidx_hbm, out_hbm,
          ids_v, flat_v, hashed_v, rowidx_v, colidx_v, rows_v,
          sem1, sem2, sem3):
        wid = lax.axis_index("s") * NC + lax.axis_index("c")
        b0 = wid * ROWS_PER_W

        pltpu.sync_copy(ids_hbm.at[pl.ds(b0, ROWS_PER_W)], ids_v)
        pltpu.sync_copy(rowidx_hbm, rowidx_v)
        pltpu.sync_copy(colidx_hbm, colidx_v)

        def repack(i, c):
            o = i * LANES
            flat_v[pl.ds(o, LANES)] = plsc.load_gather(
                ids_v, [rowidx_v[pl.ds(o, LANES)], colidx_v[pl.ds(o, LANES)]])
            return c
        lax.fori_loop(0, TOK_PER_W // LANES, repack, 0)

        def fire_lut(j, c):
            pltpu.async_copy(lut_hbm.at[flat_v.at[pl.ds(j * CHUNK, CHUNK)]],
                             hashed_v.at[pl.ds(j * CHUNK, CHUNK)], sem1)
            return c

        def s1_window(g, c):
            lax.fori_loop(g * S1G, (g + 1) * S1G, fire_lut, 0)

            @pl.when(g >= 1)
            def _():
                pltpu.make_async_copy(lut_hbm.at[pl.ds(0, S1G * CHUNK)],
                                      hashed_v.at[pl.ds(0, S1G * CHUNK)],
                                      sem1).wait()
            return c
        lax.fori_loop(0, NS1G, s1_window, 0)
        pltpu.make_async_copy(lut_hbm.at[pl.ds(0, S1G * CHUNK)],
                              hashed_v.at[pl.ds(0, S1G * CHUNK)], sem1).wait()

        def group(g, c):
            buf = lax.rem(g, 2)

            @pl.when(g >= 2)
            def _():
                pltpu.make_async_copy(table_hbm.at[pl.ds(0, GROUP_TOK)],
                                      rows_v.at[buf], sem3).wait()

            def fire_rows(j, c2):
                pltpu.async_copy(
                    table_hbm.at[hashed_v.at[pl.ds((g * GROUPC + j) * CHUNK,
                                                   CHUNK)]],
                    rows_v.at[buf].at[pl.ds(j * CHUNK, CHUNK)], sem2)
                return c2
            lax.fori_loop(0, GROUPC, fire_rows, 0)
            pltpu.make_async_copy(table_hbm.at[pl.ds(0, GROUP_TOK)],
                                  rows_v.at[buf], sem2).wait()

            def copy_row(r, c2):
                pltpu.async_copy(rows_v.at[buf].at[pl.ds(r * SEQ, SEQ)],
                                 out_hbm.at[b0 + g * GROUP_ROWS + r], sem3)
                return c2
            lax.fori_loop(0, GROUP_ROWS, copy_row, 0)
            return c
        lax.fori_loop(0, NGROUPS, group, 0)

        pltpu.make_async_copy(table_hbm.at[pl.ds(0, GROUP_TOK)], rows_v.at[0],
                              sem3).wait()
        pltpu.make_async_copy(table_hbm.at[pl.ds(0, GROUP_TOK)], rows_v.at[1],
                              sem3).wait()

    return k


_sc_lookup = _make_sc_kernel()


def kernel(input_ids, table):
    return _sc_lookup(input_ids, jnp.asarray(_LUT), table,
                      jnp.asarray(_ROWIDX), jnp.asarray(_COLIDX))
